# baseline (speedup 1.0000x reference)
import functools

import jax
import jax.numpy as jnp
from jax import lax
from jax.experimental import pallas as pl
from jax.experimental.pallas import tpu as pltpu
from jax.experimental.pallas import tpu_sc as plsc

NC = 2
NS = 16
NW = NC * NS
CHUNK = 128
NBUF = 8


@functools.partial(jax.jit, static_argnums=(1, 2, 3))
def _sc_gather(args, N, D, n_chunks):
    mesh = plsc.VectorSubcoreMesh(core_axis_name="c", subcore_axis_name="s")

    @functools.partial(
        pl.kernel,
        mesh=mesh,
        out_type=jax.ShapeDtypeStruct((N, D), jnp.float32),
        scratch_types=[
            pltpu.VMEM((n_chunks, CHUNK), jnp.int32),
            pltpu.VMEM((NBUF, CHUNK, D), jnp.float32),
        ] + [pltpu.SemaphoreType.DMA] * NBUF,
        compiler_params=pltpu.CompilerParams(use_tc_tiling_on_sc=False),
    )
    def k(idx_hbm, table_hbm, out_hbm, idx_v, rows_v, *sems):
        wid = lax.axis_index("s") * NC + lax.axis_index("c")
        base = wid * (n_chunks * CHUNK)
        pltpu.sync_copy(idx_hbm.at[wid], idx_v)

        def body(g, carry):
            c0 = g * NBUF
            handles = [
                pltpu.async_copy(
                    table_hbm.at[idx_v.at[c0 + b]], rows_v.at[b], sems[b])
                for b in range(NBUF)
            ]
            for b in range(NBUF):
                handles[b].wait()
                pltpu.sync_copy(
                    rows_v.at[b],
                    out_hbm.at[pl.ds(base + (c0 + b) * CHUNK, CHUNK)])
            return carry

        lax.fori_loop(0, n_chunks // NBUF, body, 0)

    idx, table = args
    return k(idx, table)


def kernel(x, table):
    B, L = x.shape
    V, D = table.shape
    N = B * L
    flat = x.reshape(N).astype(jnp.int32)
    group = NW * CHUNK * NBUF
    Np = ((N + group - 1) // group) * group
    if Np != N:
        flat = jnp.pad(flat, (0, Np - N))
    n_chunks = Np // (NW * CHUNK)
    idx3 = flat.reshape(NW, n_chunks, CHUNK)
    out = _sc_gather((idx3, table), Np, D, n_chunks)
    if Np != N:
        out = out[:N]
    return out.reshape(B, L, D)


# speedup vs baseline: 1.0028x; 1.0028x over previous
import functools

import jax
import jax.numpy as jnp
from jax import lax
from jax.experimental import pallas as pl
from jax.experimental.pallas import tpu as pltpu
from jax.experimental.pallas import tpu_sc as plsc

NC = 2
NS = 16
NW = NC * NS
CHUNK = 128
NBUF = 10
LAG = 3


@functools.partial(jax.jit, static_argnums=(1, 2, 3))
def _sc_gather(args, N, D, n_chunks):
    mesh = plsc.VectorSubcoreMesh(core_axis_name="c", subcore_axis_name="s")

    @functools.partial(
        pl.kernel,
        mesh=mesh,
        out_type=jax.ShapeDtypeStruct((N, D), jnp.float32),
        scratch_types=[
            pltpu.VMEM((n_chunks, CHUNK), jnp.int32),
            pltpu.VMEM((NBUF, CHUNK, D), jnp.float32),
        ] + [pltpu.SemaphoreType.DMA] * (2 * NBUF),
        compiler_params=pltpu.CompilerParams(use_tc_tiling_on_sc=False),
    )
    def k(idx_hbm, table_hbm, out_hbm, idx_v, rows_v, *sems):
        gsem = sems[:NBUF]
        ssem = sems[NBUF:]
        wid = lax.axis_index("s") * NC + lax.axis_index("c")
        base = wid * (n_chunks * CHUNK)
        pltpu.sync_copy(idx_hbm.at[wid], idx_v)

        def gather(c, b, sem):
            return pltpu.async_copy(
                table_hbm.at[idx_v.at[c]], rows_v.at[b], sem)

        def store(c, b, sem):
            return pltpu.async_copy(
                rows_v.at[b],
                out_hbm.at[pl.ds(base + c * CHUNK, CHUNK)], sem)

        for b in range(NBUF):
            gather(b, b, gsem[b])

        def body(g, carry):
            c0 = g * NBUF
            for b in range(NBUF):
                c = c0 + b
                pltpu.make_async_copy(
                    table_hbm.at[idx_v.at[c]], rows_v.at[b], gsem[b]).wait()
                store(c, b, ssem[b])
                cl = c - LAG
                b2 = (b - LAG) % NBUF
                c2 = cl + NBUF

                @pl.when(jnp.logical_and(cl >= 0, c2 < n_chunks))
                def _():
                    pltpu.make_async_copy(
                        rows_v.at[b2],
                        out_hbm.at[pl.ds(base + cl * CHUNK, CHUNK)],
                        ssem[b2]).wait()
                    gather(c2, b2, gsem[b2])

            return carry

        lax.fori_loop(0, n_chunks // NBUF, body, 0)

        for b in range(NBUF):
            c = n_chunks - NBUF + b
            pltpu.make_async_copy(
                rows_v.at[b],
                out_hbm.at[pl.ds(base + c * CHUNK, CHUNK)], ssem[b]).wait()

    idx, table = args
    return k(idx, table)


def kernel(x, table):
    B, L = x.shape
    V, D = table.shape
    N = B * L
    flat = x.reshape(N).astype(jnp.int32)
    group = NW * CHUNK * NBUF
    Np = ((N + group - 1) // group) * group
    if Np != N:
        flat = jnp.pad(flat, (0, Np - N))
    n_chunks = Np // (NW * CHUNK)
    idx3 = flat.reshape(NW, n_chunks, CHUNK)
    out = _sc_gather((idx3, table), Np, D, n_chunks)
    if Np != N:
        out = out[:N]
    return out.reshape(B, L, D)
